# baseline (speedup 1.0000x reference)
import jax
import jax.numpy as jnp
from jax import lax
from jax.experimental import pallas as pl
from jax.experimental.pallas import tpu as pltpu

_B, _T, _D, _H = 16, 512, 128, 16


_K = 4
_G = _T // _K


def _fwd_body(seq_ref, len_ref, px_ref, py_ref, out_ref, e_ref, w_ref, m_ref,
              d_ref, ph_ref):
    f32 = jnp.float32
    py = jnp.clip(py_ref[...], 1e-5, 1.0 - 1e-5)
    p0 = py[:, 0, :]
    p1 = py[:, 1, :]
    l00 = jnp.log1p(-p0)
    l01 = jnp.log(p0)
    l10 = jnp.log1p(-p1)
    l11 = jnp.log(p1)
    a_t = (l01 - l00).T.astype(jnp.bfloat16)
    b_t = (l10 - l00).T.astype(jnp.bfloat16)
    c_t = (l11 - l10 - l01 + l00).T.astype(jnp.bfloat16)
    base = jnp.sum(l00, axis=1)

    for b in range(_B):
        yb = seq_ref[b].astype(jnp.bfloat16)
        ypb = jnp.concatenate([jnp.zeros((1, _D), jnp.bfloat16), yb[:-1]], axis=0)
        eb = (jnp.dot(yb, a_t, preferred_element_type=f32)
              + jnp.dot(ypb, b_t, preferred_element_type=f32)
              + jnp.dot(yb * ypb, c_t, preferred_element_type=f32)
              + base[None, :])
        e_ref[:, b, :] = eb

    e_all = e_ref[...]
    m = jnp.max(e_all, axis=2)
    m_ref[...] = m
    w_ref[...] = jnp.exp(e_all - m[:, :, None]).astype(jnp.bfloat16)

    pmat = jnp.clip(px_ref[...], 1e-6, None).astype(jnp.bfloat16)
    p_init = (lax.broadcasted_iota(jnp.int32, (_B, _H), 1) == 0).astype(
        jnp.bfloat16)
    lens = len_ref[...]
    n_groups = (jnp.max(lens) + _K - 1) // _K

    def group(g, p):
        t0 = g * _K
        for i in range(_K - 1):
            w = w_ref[t0 + i]
            p = (jnp.dot(p, pmat, preferred_element_type=f32)
                 * w).astype(jnp.bfloat16)
            ph_ref[t0 + i] = p
        pf = jnp.dot(p, pmat, preferred_element_type=f32) * w_ref[t0 + _K - 1]
        zl = jnp.sum(pf, axis=1, keepdims=True)
        d_ref[g] = zl[:, 0]
        p = (pf / zl).astype(jnp.bfloat16)
        ph_ref[t0 + _K - 1] = p
        return p

    lax.fori_loop(0, n_groups, group, p_init)

    tt = lax.broadcasted_iota(jnp.int32, (_T, _B), 0)
    mask = tt < lens[None, :]
    msum = jnp.sum(jnp.where(mask, m_ref[...], 0.0), axis=0)
    zraw = jnp.sum(ph_ref[...].astype(f32), axis=2)
    is_renorm = (tt % _K) == (_K - 1)
    dd = jnp.repeat(d_ref[...], _K, axis=0)
    z_all = jnp.where(is_renorm, dd, zraw)
    zterm = jnp.sum(jnp.where(tt == lens[None, :] - 1, jnp.log(z_all), 0.0),
                    axis=0)
    gg = lax.broadcasted_iota(jnp.int32, (_G, _B), 0)
    dmask = (gg + 1) * _K < lens[None, :]
    dterm = jnp.sum(jnp.where(dmask, jnp.log(d_ref[...]), 0.0), axis=0)
    out_ref[...] = msum + zterm + dterm


def kernel(sequences, lengths, probs_x, probs_y):
    return pl.pallas_call(
        _fwd_body,
        out_shape=jax.ShapeDtypeStruct((_B,), jnp.float32),
        scratch_shapes=[
            pltpu.VMEM((_T, _B, _H), jnp.float32),
            pltpu.VMEM((_T, _B, _H), jnp.bfloat16),
            pltpu.VMEM((_T, _B), jnp.float32),
            pltpu.VMEM((_G, _B), jnp.float32),
            pltpu.VMEM((_T, _B, _H), jnp.bfloat16),
        ],
    )(sequences, lengths, probs_x.astype(jnp.float32), probs_y)


# speedup vs baseline: 4.2744x; 4.2744x over previous
import jax
import jax.numpy as jnp
from jax import lax
from jax.experimental import pallas as pl
from jax.experimental.pallas import tpu as pltpu

_B, _T, _D, _H = 16, 512, 128, 16


_K = 4
_G = _T // _K


def _fwd_body(seq_ref, len_ref, px_ref, py_ref, out_ref, e_ref, w_ref, m_ref,
              d_ref, z_ref):
    f32 = jnp.float32
    py = jnp.clip(py_ref[...], 1e-5, 1.0 - 1e-5)
    p0 = py[:, 0, :]
    p1 = py[:, 1, :]
    l00 = jnp.log1p(-p0)
    l01 = jnp.log(p0)
    l10 = jnp.log1p(-p1)
    l11 = jnp.log(p1)
    a_t = (l01 - l00).T.astype(jnp.bfloat16)
    b_t = (l10 - l00).T.astype(jnp.bfloat16)
    c_t = (l11 - l10 - l01 + l00).T.astype(jnp.bfloat16)
    base = jnp.sum(l00, axis=1)

    for b in range(_B):
        yb = seq_ref[b].astype(jnp.bfloat16)
        ypb = jnp.concatenate([jnp.zeros((1, _D), jnp.bfloat16), yb[:-1]], axis=0)
        eb = (jnp.dot(yb, a_t, preferred_element_type=f32)
              + jnp.dot(ypb, b_t, preferred_element_type=f32)
              + jnp.dot(yb * ypb, c_t, preferred_element_type=f32)
              + base[None, :])
        e_ref[:, b, :] = eb

    e_all = e_ref[...]
    m = jnp.max(e_all, axis=2)
    m_ref[...] = m
    w_ref[...] = jnp.swapaxes(jnp.exp(e_all - m[:, :, None]), 1, 2)

    pmat = jnp.clip(px_ref[...], 1e-6, None)
    pmt = pmat.T
    pbs = [jnp.broadcast_to(pmt[:, k:k + 1], (_H, _B)) for k in range(_H)]
    q_init = (lax.broadcasted_iota(jnp.int32, (_H, _B), 0) == 0).astype(f32)
    lens = len_ref[...]
    n_groups = (jnp.max(lens) + _K - 1) // _K

    def group(g, q):
        t0 = g * _K
        for i in range(_K):
            wt = w_ref[t0 + i]
            prods = [jnp.broadcast_to(q[k:k + 1, :], (_H, _B)) * pbs[k]
                     for k in range(_H)]
            while len(prods) > 1:
                prods = [prods[j] + prods[j + 1]
                         for j in range(0, len(prods), 2)]
            qn = prods[0] * wt
            z = jnp.sum(qn, axis=0, keepdims=True)
            z_ref[t0 + i] = z[0]
            if i == _K - 1:
                d_ref[g] = z[0]
                qn = qn / z
            q = qn
        return q

    lax.fori_loop(0, n_groups, group, q_init)

    tt = lax.broadcasted_iota(jnp.int32, (_T, _B), 0)
    mask = tt < lens[None, :]
    msum = jnp.sum(jnp.where(mask, m_ref[...], 0.0), axis=0)
    zterm = jnp.sum(jnp.where(tt == lens[None, :] - 1, jnp.log(z_ref[...]), 0.0),
                    axis=0)
    gg = lax.broadcasted_iota(jnp.int32, (_G, _B), 0)
    dmask = (gg + 1) * _K < lens[None, :]
    dterm = jnp.sum(jnp.where(dmask, jnp.log(d_ref[...]), 0.0), axis=0)
    out_ref[...] = msum + zterm + dterm


def kernel(sequences, lengths, probs_x, probs_y):
    return pl.pallas_call(
        _fwd_body,
        out_shape=jax.ShapeDtypeStruct((_B,), jnp.float32),
        scratch_shapes=[
            pltpu.VMEM((_T, _B, _H), jnp.float32),
            pltpu.VMEM((_T, _H, _B), jnp.float32),
            pltpu.VMEM((_T, _B), jnp.float32),
            pltpu.VMEM((_G, _B), jnp.float32),
            pltpu.VMEM((_T, _B), jnp.float32),
        ],
    )(sequences, lengths, probs_x.astype(jnp.float32), probs_y)


# speedup vs baseline: 4.4364x; 1.0379x over previous
import jax
import jax.numpy as jnp
from jax import lax
from jax.experimental import pallas as pl
from jax.experimental.pallas import tpu as pltpu

_B, _T, _D, _H = 16, 512, 128, 16


_K = 4
_G = _T // _K


def _fwd_body(seq_ref, len_ref, px_ref, py_ref, out_ref, e_ref, w_ref, m_ref,
              d_ref, z_ref):
    f32 = jnp.float32
    py = jnp.clip(py_ref[...], 1e-5, 1.0 - 1e-5)
    p0 = py[:, 0, :]
    p1 = py[:, 1, :]
    l00 = jnp.log1p(-p0)
    l01 = jnp.log(p0)
    l10 = jnp.log1p(-p1)
    l11 = jnp.log(p1)
    a_t = (l01 - l00).T.astype(jnp.bfloat16)
    b_t = (l10 - l00).T.astype(jnp.bfloat16)
    c_t = (l11 - l10 - l01 + l00).T.astype(jnp.bfloat16)
    base = jnp.sum(l00, axis=1)

    for b in range(_B):
        yb = seq_ref[b].astype(jnp.bfloat16)
        ypb = jnp.concatenate([jnp.zeros((1, _D), jnp.bfloat16), yb[:-1]], axis=0)
        eb = (jnp.dot(yb, a_t, preferred_element_type=f32)
              + jnp.dot(ypb, b_t, preferred_element_type=f32)
              + jnp.dot(yb * ypb, c_t, preferred_element_type=f32)
              + base[None, :])
        e_ref[:, b, :] = eb

    e_all = e_ref[...]
    m = jnp.max(e_all, axis=2)
    m_ref[...] = m
    w_ref[...] = jnp.swapaxes(jnp.exp(e_all - m[:, :, None]), 1, 2).astype(
        jnp.bfloat16)

    pmat = jnp.clip(px_ref[...], 1e-6, None)
    pmt = pmat.T.astype(jnp.bfloat16)
    pbs = [jnp.broadcast_to(pmt[:, k:k + 1], (_H, _B)) for k in range(_H)]
    q_init = (lax.broadcasted_iota(jnp.int32, (_H, _B), 0) == 0).astype(
        jnp.bfloat16)
    lens = len_ref[...]
    n_groups = (jnp.max(lens) + _K - 1) // _K

    def group(g, q):
        t0 = g * _K
        for i in range(_K):
            wt = w_ref[t0 + i]
            prods = [jnp.broadcast_to(q[k:k + 1, :], (_H, _B)) * pbs[k]
                     for k in range(_H)]
            while len(prods) > 1:
                prods = [prods[j] + prods[j + 1]
                         for j in range(0, len(prods), 2)]
            qn = prods[0] * wt
            z = jnp.sum(qn.astype(f32), axis=0, keepdims=True)
            z_ref[t0 + i] = z[0]
            if i == _K - 1:
                d_ref[g] = z[0]
                qn = (qn.astype(f32) / z).astype(jnp.bfloat16)
            q = qn
        return q

    lax.fori_loop(0, n_groups, group, q_init)

    tt = lax.broadcasted_iota(jnp.int32, (_T, _B), 0)
    mask = tt < lens[None, :]
    msum = jnp.sum(jnp.where(mask, m_ref[...], 0.0), axis=0)
    zterm = jnp.sum(jnp.where(tt == lens[None, :] - 1, jnp.log(z_ref[...]), 0.0),
                    axis=0)
    gg = lax.broadcasted_iota(jnp.int32, (_G, _B), 0)
    dmask = (gg + 1) * _K < lens[None, :]
    dterm = jnp.sum(jnp.where(dmask, jnp.log(d_ref[...]), 0.0), axis=0)
    out_ref[...] = msum + zterm + dterm


def kernel(sequences, lengths, probs_x, probs_y):
    return pl.pallas_call(
        _fwd_body,
        out_shape=jax.ShapeDtypeStruct((_B,), jnp.float32),
        scratch_shapes=[
            pltpu.VMEM((_T, _B, _H), jnp.float32),
            pltpu.VMEM((_T, _H, _B), jnp.bfloat16),
            pltpu.VMEM((_T, _B), jnp.float32),
            pltpu.VMEM((_G, _B), jnp.float32),
            pltpu.VMEM((_T, _B), jnp.float32),
        ],
    )(sequences, lengths, probs_x.astype(jnp.float32), probs_y)
